# Initial kernel scaffold; baseline (speedup 1.0000x reference)
import jax
import jax.numpy as jnp
from jax.experimental import pallas as pl


def kernel(x, edge_index, W_in, b_in, W_cheb, b_cheb, W_out, b_out):
    raise NotImplementedError("write your pallas kernel here")


# speedup vs baseline: 5.6671x; 5.6671x over previous
import functools

import jax
import jax.numpy as jnp
from jax import lax
from jax.experimental import pallas as pl
from jax.experimental.pallas import tpu as pltpu
from jax.experimental.pallas import tpu_sc as plsc

N = 10000
E = 320000
D_IN = 128
H = 128
C = 2

NC = 2
NS = 16
NW = NC * NS
EPW = E // NW
CH = 125
NCH = EPW // CH
NP = 10240
RPT = NP // NS
ZCH = 80
DEGW = 128

_mesh = plsc.VectorSubcoreMesh(core_axis_name="c", subcore_axis_name="s")


@functools.partial(
    pl.kernel,
    out_type=jax.ShapeDtypeStruct((NC, NP, DEGW), jnp.float32),
    mesh=_mesh,
    scratch_types=[
        pltpu.VMEM_SHARED((NP, DEGW), jnp.float32),
        pltpu.VMEM((NCH, CH), jnp.int32),
        pltpu.VMEM((CH, DEGW), jnp.float32),
        pltpu.VMEM((ZCH, DEGW), jnp.float32),
    ],
)
def _sc_degree(dst_hbm, out_hbm, acc, didx, ones_v, zero_v):
    c = lax.axis_index("c")
    s = lax.axis_index("s")
    w = c * NS + s
    one16 = jnp.full((16,), 1.0, jnp.float32)
    zer16 = jnp.zeros((16,), jnp.float32)
    dw = DEGW // 16

    def fill(i, carry):
        ones_v[i // dw, pl.ds((i % dw) * 16, 16)] = one16
        return carry

    lax.fori_loop(0, CH * dw, fill, 0)

    def fillz(i, carry):
        zero_v[i // dw, pl.ds((i % dw) * 16, 16)] = zer16
        return carry

    lax.fori_loop(0, ZCH * dw, fillz, 0)

    base = s * RPT

    def zc(k, carry):
        pltpu.sync_copy(zero_v, acc.at[pl.ds(base + k * ZCH, ZCH)])
        return carry

    lax.fori_loop(0, RPT // ZCH, zc, 0)
    plsc.subcore_barrier()

    pltpu.sync_copy(dst_hbm.at[pl.ds(w * NCH, NCH)], didx)

    def step(j, carry):
        pltpu.sync_copy(ones_v, acc.at[didx.at[j]], add=True)
        return carry

    lax.fori_loop(0, NCH, step, 0)
    plsc.subcore_barrier()
    pltpu.sync_copy(acc.at[pl.ds(base, RPT)], out_hbm.at[c].at[pl.ds(base, RPT)])


@functools.partial(
    pl.kernel,
    out_type=jax.ShapeDtypeStruct((NC, NP, H), jnp.float32),
    mesh=_mesh,
    scratch_types=[
        pltpu.VMEM_SHARED((NP, H), jnp.float32),
        pltpu.VMEM((NCH, CH), jnp.int32),
        pltpu.VMEM((NCH, CH), jnp.int32),
        pltpu.VMEM((CH, H), jnp.float32),
        pltpu.SemaphoreType.DMA,
    ],
)
def _sc_scatter(g_hbm, src_hbm, dst_hbm, out_hbm, acc, sidx, didx, rowbuf, sem):
    c = lax.axis_index("c")
    s = lax.axis_index("s")
    w = c * NS + s
    zer16 = jnp.zeros((16,), jnp.float32)
    hb = H // 16

    def zb(i, carry):
        rowbuf[i // hb, pl.ds((i % hb) * 16, 16)] = zer16
        return carry

    lax.fori_loop(0, CH * hb, zb, 0)

    base = s * RPT

    def zc(k, carry):
        pltpu.sync_copy(rowbuf.at[pl.ds(0, ZCH)], acc.at[pl.ds(base + k * ZCH, ZCH)])
        return carry

    lax.fori_loop(0, RPT // ZCH, zc, 0)
    plsc.subcore_barrier()

    pltpu.sync_copy(src_hbm.at[pl.ds(w * NCH, NCH)], sidx)
    pltpu.sync_copy(dst_hbm.at[pl.ds(w * NCH, NCH)], didx)

    def step(j, carry):
        pltpu.sync_copy(g_hbm.at[sidx.at[j]], rowbuf)
        pltpu.sync_copy(rowbuf, acc.at[didx.at[j]], add=True)
        return carry

    lax.fori_loop(0, NCH, step, 0)
    plsc.subcore_barrier()
    pltpu.sync_copy(acc.at[pl.ds(base, RPT)], out_hbm.at[c].at[pl.ds(base, RPT)])


def _tc1_body(deg_ref, x_ref, w_ref, b_ref, g_ref, nrm_ref, inv_ref):
    d = deg_ref[0, :, :1] + deg_ref[1, :, :1]
    degc = jnp.maximum(d, 1.0)
    nrm = lax.rsqrt(degc)
    hv = jnp.dot(x_ref[...], w_ref[...], preferred_element_type=jnp.float32)
    hv = jnp.maximum(hv + b_ref[...], 0.0)
    g_ref[...] = hv * nrm
    nrm_ref[...] = nrm
    inv_ref[...] = jnp.sqrt(degc)


_R = 1000


def _tc_stage1(deg_parts, x, w_in_t, b_in):
    grid = (N // _R,)
    return pl.pallas_call(
        _tc1_body,
        grid=grid,
        in_specs=[
            pl.BlockSpec((NC, _R, DEGW), lambda i: (0, i, 0)),
            pl.BlockSpec((_R, D_IN), lambda i: (i, 0)),
            pl.BlockSpec((D_IN, H), lambda i: (0, 0)),
            pl.BlockSpec((1, H), lambda i: (0, 0)),
        ],
        out_specs=[
            pl.BlockSpec((_R, H), lambda i: (i, 0)),
            pl.BlockSpec((_R, 1), lambda i: (i, 0)),
            pl.BlockSpec((_R, 1), lambda i: (i, 0)),
        ],
        out_shape=[
            jax.ShapeDtypeStruct((N, H), jnp.float32),
            jax.ShapeDtypeStruct((N, 1), jnp.float32),
            jax.ShapeDtypeStruct((N, 1), jnp.float32),
        ],
    )(deg_parts, x, w_in_t, b_in)


def _tc2_body(g_ref, sp_ref, nrm_ref, inv_ref, w1_ref, w2_ref, bc_ref,
              wo_ref, bo_ref, out_ref):
    sm = sp_ref[0] + sp_ref[1]
    p = jnp.dot(g_ref[...], w1_ref[...], preferred_element_type=jnp.float32)
    q = jnp.dot(sm, w2_ref[...], preferred_element_type=jnp.float32)
    h2 = jnp.maximum(p * inv_ref[...] - q * nrm_ref[...] + bc_ref[...], 0.0)
    out_ref[...] = (
        jnp.dot(h2, wo_ref[...], preferred_element_type=jnp.float32)
        + bo_ref[...]
    )


def _tc_stage2(g, s_parts, nrm, inv, w1_t, w2_t, b_cheb, w_out_t, b_out):
    grid = (N // _R,)
    return pl.pallas_call(
        _tc2_body,
        grid=grid,
        in_specs=[
            pl.BlockSpec((_R, H), lambda i: (i, 0)),
            pl.BlockSpec((NC, _R, H), lambda i: (0, i, 0)),
            pl.BlockSpec((_R, 1), lambda i: (i, 0)),
            pl.BlockSpec((_R, 1), lambda i: (i, 0)),
            pl.BlockSpec((H, H), lambda i: (0, 0)),
            pl.BlockSpec((H, H), lambda i: (0, 0)),
            pl.BlockSpec((1, H), lambda i: (0, 0)),
            pl.BlockSpec((H, C), lambda i: (0, 0)),
            pl.BlockSpec((1, C), lambda i: (0, 0)),
        ],
        out_specs=pl.BlockSpec((_R, C), lambda i: (i, 0)),
        out_shape=jax.ShapeDtypeStruct((N, C), jnp.float32),
    )(g, s_parts, nrm, inv, w1_t, w2_t, b_cheb, w_out_t, b_out)


def kernel(x, edge_index, W_in, b_in, W_cheb, b_cheb, W_out, b_out):
    src2 = edge_index[0].reshape(E // CH, CH)
    dst2 = edge_index[1].reshape(E // CH, CH)

    deg_parts = _sc_degree(dst2)
    g, nrm, inv = _tc_stage1(deg_parts, x, W_in.T, b_in.reshape(1, H))
    s_parts = _sc_scatter(g, src2, dst2)
    out = _tc_stage2(
        g, s_parts, nrm, inv,
        W_cheb[:, :H].T, W_cheb[:, H:].T, b_cheb.reshape(1, H),
        W_out.T, b_out.reshape(1, C),
    )
    return out


# speedup vs baseline: 8.0794x; 1.4257x over previous
import functools

import jax
import jax.numpy as jnp
from jax import lax
from jax.experimental import pallas as pl
from jax.experimental.pallas import tpu as pltpu
from jax.experimental.pallas import tpu_sc as plsc

N = 10000
E = 320000
D_IN = 128
H = 128
C = 2

NC = 2
NS = 16
NW = NC * NS
EPW = E // NW
CH = 125
NCH = EPW // CH
NP = 10240
RPT = NP // NS
ZCH = 80

_mesh = plsc.VectorSubcoreMesh(core_axis_name="c", subcore_axis_name="s")


@functools.partial(
    pl.kernel,
    out_type=jax.ShapeDtypeStruct((NC, NP * 8), jnp.float32),
    mesh=_mesh,
    compiler_params=pltpu.CompilerParams(needs_layout_passes=False),
    scratch_types=[
        pltpu.VMEM_SHARED((NS * NP,), jnp.float32),
        pltpu.VMEM((EPW,), jnp.int32),
        pltpu.VMEM((NP,), jnp.float32),
        pltpu.VMEM((RPT,), jnp.float32),
        pltpu.VMEM((RPT,), jnp.float32),
        pltpu.VMEM((RPT * 8,), jnp.float32),
    ],
)
def _sc_degree(dst_hbm, out_hbm, stage, idx_v, hist_v, acc_v, tmp_v, obuf):
    c = lax.axis_index("c")
    s = lax.axis_index("s")
    w = c * NS + s
    one16 = jnp.full((16,), 1.0, jnp.float32)
    zer16 = jnp.zeros((16,), jnp.float32)

    def zh(i, carry):
        hist_v[pl.ds(i * 16, 16)] = zer16
        return carry

    lax.fori_loop(0, NP // 16, zh, 0)
    pltpu.sync_copy(dst_hbm.at[pl.ds(w * EPW, EPW)], idx_v)

    def step(j, carry):
        iv = idx_v[pl.ds(j * 16, 16)]
        plsc.addupdate_scatter(hist_v, [iv], one16)
        return carry

    lax.fori_loop(0, EPW // 16, step, 0)
    pltpu.sync_copy(hist_v, stage.at[pl.ds(s * NP, NP)])
    plsc.subcore_barrier()

    base = s * RPT

    def za(i, carry):
        acc_v[pl.ds(i * 16, 16)] = zer16
        return carry

    lax.fori_loop(0, RPT // 16, za, 0)

    def red(t, carry):
        pltpu.sync_copy(stage.at[pl.ds(t * NP + base, RPT)], tmp_v)

        def add(i, carry2):
            acc_v[pl.ds(i * 16, 16)] = (
                acc_v[pl.ds(i * 16, 16)] + tmp_v[pl.ds(i * 16, 16)]
            )
            return carry2

        lax.fori_loop(0, RPT // 16, add, 0)
        return carry

    lax.fori_loop(0, NS, red, 0)

    def put(k, carry):
        rows = (lax.iota(jnp.int32, 16) + k * 16) * 8
        plsc.store_scatter(obuf, [rows], acc_v[pl.ds(k * 16, 16)])
        return carry

    lax.fori_loop(0, RPT // 16, put, 0)
    pltpu.sync_copy(obuf, out_hbm.at[c, pl.ds(base * 8, RPT * 8)])


@functools.partial(
    pl.kernel,
    out_type=jax.ShapeDtypeStruct((NC, NP, H), jnp.float32),
    mesh=_mesh,
    scratch_types=[
        pltpu.VMEM_SHARED((NP, H), jnp.float32),
        pltpu.VMEM((CH,), jnp.int32),
        pltpu.VMEM((CH,), jnp.int32),
        pltpu.VMEM((CH,), jnp.int32),
        pltpu.VMEM((CH,), jnp.int32),
        pltpu.VMEM((CH, H), jnp.float32),
        pltpu.VMEM((CH, H), jnp.float32),
        pltpu.SemaphoreType.DMA,
        pltpu.SemaphoreType.DMA,
        pltpu.SemaphoreType.DMA,
        pltpu.SemaphoreType.DMA,
    ],
)
def _sc_scatter(g_hbm, src_hbm, dst_hbm, out_hbm, acc, sa0, sa1, da0, da1,
                buf0, buf1, sem0, sem1, semi0, semi1):
    c = lax.axis_index("c")
    s = lax.axis_index("s")
    w = c * NS + s
    zer16 = jnp.zeros((16,), jnp.float32)
    hb = H // 16

    def zb(i, carry):
        buf0[i // hb, pl.ds((i % hb) * 16, 16)] = zer16
        return carry

    lax.fori_loop(0, CH * hb, zb, 0)

    base = s * RPT

    def zc(k, carry):
        pltpu.sync_copy(buf0.at[pl.ds(0, ZCH)], acc.at[pl.ds(base + k * ZCH, ZCH)])
        return carry

    lax.fori_loop(0, RPT // ZCH, zc, 0)
    plsc.subcore_barrier()

    row0 = w * NCH

    pltpu.sync_copy(src_hbm.at[row0], sa0)
    pltpu.sync_copy(dst_hbm.at[row0], da0)
    pltpu.async_copy(g_hbm.at[sa0], buf0, sem0)
    pltpu.async_copy(src_hbm.at[row0 + 1], sa1, semi1)
    pltpu.async_copy(dst_hbm.at[row0 + 1], da1, semi1)
    nk2 = NCH // 2

    def step2(k, carry):
        j0 = row0 + 2 * k
        j1 = j0 + 1
        pltpu.make_async_copy(g_hbm.at[sa0], buf0, sem0).wait()
        pltpu.make_async_copy(src_hbm.at[j1], sa1, semi1).wait()
        pltpu.make_async_copy(dst_hbm.at[j1], da1, semi1).wait()
        pltpu.async_copy(g_hbm.at[sa1], buf1, sem1)
        pltpu.sync_copy(buf0, acc.at[da0], add=True)

        @pl.when(k < nk2 - 1)
        def _():
            pltpu.async_copy(src_hbm.at[j0 + 2], sa0, semi0)
            pltpu.async_copy(dst_hbm.at[j0 + 2], da0, semi0)

        pltpu.make_async_copy(g_hbm.at[sa1], buf1, sem1).wait()

        @pl.when(k < nk2 - 1)
        def _():
            pltpu.make_async_copy(src_hbm.at[j0 + 2], sa0, semi0).wait()
            pltpu.make_async_copy(dst_hbm.at[j0 + 2], da0, semi0).wait()
            pltpu.async_copy(g_hbm.at[sa0], buf0, sem0)

        pltpu.sync_copy(buf1, acc.at[da1], add=True)

        @pl.when(k < nk2 - 1)
        def _():
            pltpu.async_copy(src_hbm.at[j1 + 2], sa1, semi1)
            pltpu.async_copy(dst_hbm.at[j1 + 2], da1, semi1)

        return carry

    lax.fori_loop(0, nk2, step2, 0)
    plsc.subcore_barrier()
    pltpu.sync_copy(acc.at[pl.ds(base, RPT)], out_hbm.at[c].at[pl.ds(base, RPT)])


def _tc1_body(deg_ref, x_ref, w_ref, b_ref, g_ref, nrm_ref, inv_ref):
    d = deg_ref[0, :, :1] + deg_ref[1, :, :1]
    degc = jnp.maximum(d, 1.0)
    nrm = lax.rsqrt(degc)
    hv = jnp.dot(x_ref[...], w_ref[...], preferred_element_type=jnp.float32)
    hv = jnp.maximum(hv + b_ref[...], 0.0)
    g_ref[...] = hv * nrm
    nrm_ref[...] = nrm
    inv_ref[...] = jnp.sqrt(degc)


_R = 1000


def _tc_stage1(deg_parts, x, w_in_t, b_in):
    grid = (N // _R,)
    return pl.pallas_call(
        _tc1_body,
        grid=grid,
        in_specs=[
            pl.BlockSpec((NC, _R, 8), lambda i: (0, i, 0)),
            pl.BlockSpec((_R, D_IN), lambda i: (i, 0)),
            pl.BlockSpec((D_IN, H), lambda i: (0, 0)),
            pl.BlockSpec((1, H), lambda i: (0, 0)),
        ],
        out_specs=[
            pl.BlockSpec((_R, H), lambda i: (i, 0)),
            pl.BlockSpec((_R, 1), lambda i: (i, 0)),
            pl.BlockSpec((_R, 1), lambda i: (i, 0)),
        ],
        out_shape=[
            jax.ShapeDtypeStruct((N, H), jnp.float32),
            jax.ShapeDtypeStruct((N, 1), jnp.float32),
            jax.ShapeDtypeStruct((N, 1), jnp.float32),
        ],
    )(deg_parts, x, w_in_t, b_in)


def _tc2_body(g_ref, sp_ref, nrm_ref, inv_ref, w1_ref, w2_ref, bc_ref,
              wo_ref, bo_ref, out_ref):
    sm = sp_ref[0] + sp_ref[1]
    p = jnp.dot(g_ref[...], w1_ref[...], preferred_element_type=jnp.float32)
    q = jnp.dot(sm, w2_ref[...], preferred_element_type=jnp.float32)
    h2 = jnp.maximum(p * inv_ref[...] - q * nrm_ref[...] + bc_ref[...], 0.0)
    out_ref[...] = (
        jnp.dot(h2, wo_ref[...], preferred_element_type=jnp.float32)
        + bo_ref[...]
    )


def _tc_stage2(g, s_parts, nrm, inv, w1_t, w2_t, b_cheb, w_out_t, b_out):
    grid = (N // _R,)
    return pl.pallas_call(
        _tc2_body,
        grid=grid,
        in_specs=[
            pl.BlockSpec((_R, H), lambda i: (i, 0)),
            pl.BlockSpec((NC, _R, H), lambda i: (0, i, 0)),
            pl.BlockSpec((_R, 1), lambda i: (i, 0)),
            pl.BlockSpec((_R, 1), lambda i: (i, 0)),
            pl.BlockSpec((H, H), lambda i: (0, 0)),
            pl.BlockSpec((H, H), lambda i: (0, 0)),
            pl.BlockSpec((1, H), lambda i: (0, 0)),
            pl.BlockSpec((H, C), lambda i: (0, 0)),
            pl.BlockSpec((1, C), lambda i: (0, 0)),
        ],
        out_specs=pl.BlockSpec((_R, C), lambda i: (i, 0)),
        out_shape=jax.ShapeDtypeStruct((N, C), jnp.float32),
    )(g, s_parts, nrm, inv, w1_t, w2_t, b_cheb, w_out_t, b_out)


def kernel(x, edge_index, W_in, b_in, W_cheb, b_cheb, W_out, b_out):
    src2 = edge_index[0].reshape(E // CH, CH)
    dst2 = edge_index[1].reshape(E // CH, CH)

    deg_parts = _sc_degree(edge_index[1]).reshape(NC, NP, 8)
    g, nrm, inv = _tc_stage1(deg_parts, x, W_in.T, b_in.reshape(1, H))
    s_parts = _sc_scatter(g, src2, dst2)
    out = _tc_stage2(
        g, s_parts, nrm, inv,
        W_cheb[:, :H].T, W_cheb[:, H:].T, b_cheb.reshape(1, H),
        W_out.T, b_out.reshape(1, C),
    )
    return out


# speedup vs baseline: 8.2472x; 1.0208x over previous
import functools

import jax
import jax.numpy as jnp
from jax import lax
from jax.experimental import pallas as pl
from jax.experimental.pallas import tpu as pltpu
from jax.experimental.pallas import tpu_sc as plsc

N = 10000
E = 320000
D_IN = 128
H = 128
C = 2

NC = 2
NS = 16
NW = NC * NS
EPW = E // NW
CH = 125
NCH = EPW // CH
NP = 10240
RPT = NP // NS
ZCH = 80

_mesh = plsc.VectorSubcoreMesh(core_axis_name="c", subcore_axis_name="s")


@functools.partial(
    pl.kernel,
    out_type=jax.ShapeDtypeStruct((NC, NP * 8), jnp.float32),
    mesh=_mesh,
    compiler_params=pltpu.CompilerParams(needs_layout_passes=False),
    scratch_types=[
        pltpu.VMEM_SHARED((NS * NP,), jnp.float32),
        pltpu.VMEM((EPW,), jnp.int32),
        pltpu.VMEM((NP,), jnp.float32),
        pltpu.VMEM((RPT,), jnp.float32),
        pltpu.VMEM((RPT,), jnp.float32),
        pltpu.VMEM((RPT * 8,), jnp.float32),
    ],
)
def _sc_degree(dst_hbm, out_hbm, stage, idx_v, hist_v, acc_v, tmp_v, obuf):
    c = lax.axis_index("c")
    s = lax.axis_index("s")
    w = c * NS + s
    one16 = jnp.full((16,), 1.0, jnp.float32)
    zer16 = jnp.zeros((16,), jnp.float32)

    def zh(i, carry):
        hist_v[pl.ds(i * 16, 16)] = zer16
        return carry

    lax.fori_loop(0, NP // 16, zh, 0)
    pltpu.sync_copy(dst_hbm.at[pl.ds(w * EPW, EPW)], idx_v)

    def step(j, carry):
        iv = idx_v[pl.ds(j * 16, 16)]
        plsc.addupdate_scatter(hist_v, [iv], one16)
        return carry

    lax.fori_loop(0, EPW // 16, step, 0)
    pltpu.sync_copy(hist_v, stage.at[pl.ds(s * NP, NP)])
    plsc.subcore_barrier()

    base = s * RPT

    def za(i, carry):
        acc_v[pl.ds(i * 16, 16)] = zer16
        return carry

    lax.fori_loop(0, RPT // 16, za, 0)

    def red(t, carry):
        pltpu.sync_copy(stage.at[pl.ds(t * NP + base, RPT)], tmp_v)

        def add(i, carry2):
            acc_v[pl.ds(i * 16, 16)] = (
                acc_v[pl.ds(i * 16, 16)] + tmp_v[pl.ds(i * 16, 16)]
            )
            return carry2

        lax.fori_loop(0, RPT // 16, add, 0)
        return carry

    lax.fori_loop(0, NS, red, 0)

    def put(k, carry):
        rows = (lax.iota(jnp.int32, 16) + k * 16) * 8
        plsc.store_scatter(obuf, [rows], acc_v[pl.ds(k * 16, 16)])
        return carry

    lax.fori_loop(0, RPT // 16, put, 0)
    pltpu.sync_copy(obuf, out_hbm.at[c, pl.ds(base * 8, RPT * 8)])


@functools.partial(
    pl.kernel,
    out_type=jax.ShapeDtypeStruct((NC, NP, H), jnp.float32),
    mesh=_mesh,
    scratch_types=[
        pltpu.VMEM_SHARED((NP, H), jnp.float32),
        pltpu.VMEM((CH,), jnp.int32),
        pltpu.VMEM((CH,), jnp.int32),
        pltpu.VMEM((CH,), jnp.int32),
        pltpu.VMEM((CH,), jnp.int32),
        pltpu.VMEM((CH, H), jnp.float32),
        pltpu.VMEM((CH, H), jnp.float32),
        pltpu.SemaphoreType.DMA,
        pltpu.SemaphoreType.DMA,
        pltpu.SemaphoreType.DMA,
        pltpu.SemaphoreType.DMA,
    ],
)
def _sc_scatter(g_hbm, src_hbm, dst_hbm, out_hbm, acc, sa0, sa1, da0, da1,
                buf0, buf1, sem0, sem1, semi0, semi1):
    c = lax.axis_index("c")
    s = lax.axis_index("s")
    w = c * NS + s
    zer16 = jnp.zeros((16,), jnp.float32)
    hb = H // 16

    def zb(i, carry):
        buf0[i // hb, pl.ds((i % hb) * 16, 16)] = zer16
        return carry

    lax.fori_loop(0, CH * hb, zb, 0)

    base = s * RPT

    def zc(k, carry):
        pltpu.sync_copy(buf0.at[pl.ds(0, ZCH)], acc.at[pl.ds(base + k * ZCH, ZCH)])
        return carry

    lax.fori_loop(0, RPT // ZCH, zc, 0)
    plsc.subcore_barrier()

    row0 = w * NCH

    pltpu.sync_copy(src_hbm.at[row0], sa0)
    pltpu.sync_copy(dst_hbm.at[row0], da0)
    pltpu.async_copy(g_hbm.at[sa0], buf0, sem0)
    pltpu.async_copy(src_hbm.at[row0 + 1], sa1, semi1)
    pltpu.async_copy(dst_hbm.at[row0 + 1], da1, semi1)
    nk2 = NCH // 2

    def step2(k, carry):
        j0 = row0 + 2 * k
        j1 = j0 + 1
        pltpu.make_async_copy(g_hbm.at[sa0], buf0, sem0).wait()
        pltpu.make_async_copy(src_hbm.at[j1], sa1, semi1).wait()
        pltpu.make_async_copy(dst_hbm.at[j1], da1, semi1).wait()
        pltpu.async_copy(g_hbm.at[sa1], buf1, sem1)
        pltpu.sync_copy(buf0, acc.at[da0], add=True)

        @pl.when(k < nk2 - 1)
        def _():
            pltpu.async_copy(src_hbm.at[j0 + 2], sa0, semi0)
            pltpu.async_copy(dst_hbm.at[j0 + 2], da0, semi0)

        pltpu.make_async_copy(g_hbm.at[sa1], buf1, sem1).wait()

        @pl.when(k < nk2 - 1)
        def _():
            pltpu.make_async_copy(src_hbm.at[j0 + 2], sa0, semi0).wait()
            pltpu.make_async_copy(dst_hbm.at[j0 + 2], da0, semi0).wait()
            pltpu.async_copy(g_hbm.at[sa0], buf0, sem0)

        pltpu.sync_copy(buf1, acc.at[da1], add=True)

        @pl.when(k < nk2 - 1)
        def _():
            pltpu.async_copy(src_hbm.at[j1 + 2], sa1, semi1)
            pltpu.async_copy(dst_hbm.at[j1 + 2], da1, semi1)

        return carry

    lax.fori_loop(0, nk2, step2, 0)
    plsc.subcore_barrier()
    pltpu.sync_copy(acc.at[pl.ds(base, RPT)], out_hbm.at[c].at[pl.ds(base, RPT)])


def _tc1_body(deg_ref, x_ref, w_ref, b_ref, g_ref, nrm_ref, inv_ref):
    d = deg_ref[0, :, :1] + deg_ref[1, :, :1]
    degc = jnp.maximum(d, 1.0)
    nrm = lax.rsqrt(degc)
    hv = lax.dot_general(
        x_ref[...], w_ref[...], (((1,), (1,)), ((), ())),
        preferred_element_type=jnp.float32)
    hv = jnp.maximum(hv + b_ref[...], 0.0)
    g_ref[...] = hv * nrm
    nrm_ref[...] = nrm
    inv_ref[...] = jnp.sqrt(degc)


_R = 2000


def _tc_stage1(deg_parts, x, w_in_t, b_in):
    grid = (N // _R,)
    return pl.pallas_call(
        _tc1_body,
        grid=grid,
        in_specs=[
            pl.BlockSpec((NC, _R, 8), lambda i: (0, i, 0)),
            pl.BlockSpec((_R, D_IN), lambda i: (i, 0)),
            pl.BlockSpec((H, D_IN), lambda i: (0, 0)),
            pl.BlockSpec((1, H), lambda i: (0, 0)),
        ],
        out_specs=[
            pl.BlockSpec((_R, H), lambda i: (i, 0)),
            pl.BlockSpec((_R, 1), lambda i: (i, 0)),
            pl.BlockSpec((_R, 1), lambda i: (i, 0)),
        ],
        out_shape=[
            jax.ShapeDtypeStruct((N, H), jnp.float32),
            jax.ShapeDtypeStruct((N, 1), jnp.float32),
            jax.ShapeDtypeStruct((N, 1), jnp.float32),
        ],
    )(deg_parts, x, w_in_t, b_in)


def _tc2_body(g_ref, sp_ref, nrm_ref, inv_ref, w1_ref, w2_ref, bc_ref,
              wo_ref, bo_ref, out_ref):
    sm = sp_ref[0] + sp_ref[1]
    dn = (((1,), (1,)), ((), ()))
    p = lax.dot_general(g_ref[...], w1_ref[...], dn,
                        preferred_element_type=jnp.float32)
    q = lax.dot_general(sm, w2_ref[...], dn,
                        preferred_element_type=jnp.float32)
    h2 = jnp.maximum(p * inv_ref[...] - q * nrm_ref[...] + bc_ref[...], 0.0)
    out_ref[...] = (
        lax.dot_general(h2, wo_ref[...], dn,
                        preferred_element_type=jnp.float32)
        + bo_ref[...]
    )


def _tc_stage2(g, s_parts, nrm, inv, w1_t, w2_t, b_cheb, w_out_t, b_out):
    grid = (N // _R,)
    return pl.pallas_call(
        _tc2_body,
        grid=grid,
        in_specs=[
            pl.BlockSpec((_R, H), lambda i: (i, 0)),
            pl.BlockSpec((NC, _R, H), lambda i: (0, i, 0)),
            pl.BlockSpec((_R, 1), lambda i: (i, 0)),
            pl.BlockSpec((_R, 1), lambda i: (i, 0)),
            pl.BlockSpec((H, H), lambda i: (0, 0)),
            pl.BlockSpec((H, H), lambda i: (0, 0)),
            pl.BlockSpec((1, H), lambda i: (0, 0)),
            pl.BlockSpec((C, H), lambda i: (0, 0)),
            pl.BlockSpec((1, C), lambda i: (0, 0)),
        ],
        out_specs=pl.BlockSpec((_R, C), lambda i: (i, 0)),
        out_shape=jax.ShapeDtypeStruct((N, C), jnp.float32),
    )(g, s_parts, nrm, inv, w1_t, w2_t, b_cheb, w_out_t, b_out)


def kernel(x, edge_index, W_in, b_in, W_cheb, b_cheb, W_out, b_out):
    src2 = edge_index[0].reshape(E // CH, CH)
    dst2 = edge_index[1].reshape(E // CH, CH)

    deg_parts = _sc_degree(edge_index[1]).reshape(NC, NP, 8)
    g, nrm, inv = _tc_stage1(deg_parts, x, W_in, b_in.reshape(1, H))
    s_parts = _sc_scatter(g, src2, dst2)
    out = _tc_stage2(
        g, s_parts, nrm, inv,
        W_cheb[:, :H], W_cheb[:, H:], b_cheb.reshape(1, H),
        W_out, b_out.reshape(1, C),
    )
    return out
